# baseline (speedup 1.0000x reference)
import functools

import jax
import jax.numpy as jnp
from jax import lax
from jax.experimental import pallas as pl
from jax.experimental.pallas import tpu as pltpu
from jax.experimental.pallas import tpu_sc as plsc

N_NODES = 10000
N_PAD = 10240
E_EDGES = 160000
E_PAD = 163840
N_TILES = 32
CHUNKS = 40
CHUNK = 128
ROWS_PER_TILE = N_PAD // 16
NEG_INF = -3.0e38


def _stage1_body(x_ref, wl_ref, wr_ref, t1_ref, xr_ref):
    xb = x_ref[...]
    xl = jnp.dot(xb, wl_ref[...], preferred_element_type=jnp.float32)
    cols = lax.broadcasted_iota(jnp.int32, (400, 32), 1)
    t1_ref[...] = (jnp.concatenate([xl, jnp.zeros((400, 16), jnp.float32)],
                                   axis=1)
                   + jnp.where(cols == 16, 1.0, 0.0))
    xr_ref[...] = jnp.dot(xb, wr_ref[...], preferred_element_type=jnp.float32)


def _stage1(x, wl, wr):
    return pl.pallas_call(
        _stage1_body,
        grid=(N_NODES // 400,),
        in_specs=[
            pl.BlockSpec((400, 256), lambda i: (i, 0)),
            pl.BlockSpec((256, 16), lambda i: (0, 0)),
            pl.BlockSpec((256, 16), lambda i: (0, 0)),
        ],
        out_specs=[
            pl.BlockSpec((400, 32), lambda i: (i, 0)),
            pl.BlockSpec((400, 16), lambda i: (i, 0)),
        ],
        out_shape=[
            jax.ShapeDtypeStruct((N_PAD, 32), jnp.float32),
            jax.ShapeDtypeStruct((N_PAD, 16), jnp.float32),
        ],
    )(x, wl, wr)


@functools.lru_cache(maxsize=None)
def _make_sc_agg(width):
    mesh = plsc.VectorSubcoreMesh(core_axis_name="c", subcore_axis_name="s")

    @functools.partial(
        pl.kernel,
        mesh=mesh,
        out_type=jax.ShapeDtypeStruct((2, N_PAD, width), jnp.float32),
        scratch_types=[
            pltpu.VMEM((2, CHUNKS, CHUNK), jnp.int32),
            [pltpu.VMEM((CHUNK, width), jnp.float32) for _ in range(2)],
            pltpu.VMEM_SHARED((N_PAD, width), jnp.float32),
            [pltpu.SemaphoreType.DMA for _ in range(2)],
        ],
        compiler_params=pltpu.CompilerParams(use_tc_tiling_on_sc=False),
    )
    def sc_agg(table_hbm, edge_hbm, zero_hbm, out_hbm,
               idx_v, rows, acc_sh, gsem):
        c = lax.axis_index("c")
        s = lax.axis_index("s")
        wid = s * 2 + c
        row0 = s * ROWS_PER_TILE
        pltpu.sync_copy(zero_hbm.at[pl.ds(row0, ROWS_PER_TILE)],
                        acc_sh.at[pl.ds(row0, ROWS_PER_TILE)])
        pltpu.sync_copy(edge_hbm.at[pl.ds(0, 2), wid], idx_v)
        plsc.subcore_barrier()
        src_v = idx_v.at[0]
        dst_v = idx_v.at[1]

        for b in range(2):
            pltpu.async_copy(table_hbm.at[src_v.at[b]], rows[b], gsem[b])

        def body(i, carry):
            for b in range(2):
                j = 2 * i + b
                pltpu.make_async_copy(table_hbm.at[src_v.at[0]],
                                      rows[b], gsem[b]).wait()
                pltpu.sync_copy(rows[b], acc_sh.at[dst_v.at[j]], add=True)

                @pl.when(j + 2 < CHUNKS)
                def _prefetch():
                    pltpu.async_copy(table_hbm.at[src_v.at[j + 2]],
                                     rows[b], gsem[b])
            return carry

        lax.fori_loop(0, CHUNKS // 2, body, 0)
        plsc.subcore_barrier()
        pltpu.sync_copy(acc_sh.at[pl.ds(row0, ROWS_PER_TILE)],
                        out_hbm.at[c, pl.ds(row0, ROWS_PER_TILE)])

    return sc_agg


def _sc_agg(table, edge4, zero, width):
    return _make_sc_agg(width)(table, edge4, zero)


def _stage2_body(a0_ref, a1_ref, xr_ref, w2l_ref, w2r_ref, b1_ref, g1_ref,
                 be1_ref, b2_ref, t2_ref, r2_ref, inv_ref):
    i = pl.program_id(0)
    acc = a0_ref[...] + a1_ref[...]
    agg = acc[:, :16]
    deg = acc[:, 16:17]
    invd = 1.0 / jnp.maximum(deg, 1.0)
    pre = agg * invd + b1_ref[...] + xr_ref[...]
    h = jnp.maximum(pre, 0.0)
    mu = jnp.mean(h, axis=1, keepdims=True)
    var = jnp.mean((h - mu) ** 2, axis=1, keepdims=True)
    hn = (h - mu) * lax.rsqrt(var + 1e-5) * g1_ref[...] + be1_ref[...]
    rows = i * 256 + lax.broadcasted_iota(jnp.int32, (256, 1), 0)
    valid = rows < N_NODES
    hl = jnp.dot(hn, w2l_ref[...], preferred_element_type=jnp.float32)
    t2_ref[...] = jnp.where(valid, hl, 0.0)
    r2_ref[...] = jnp.dot(hn, w2r_ref[...],
                          preferred_element_type=jnp.float32) + b2_ref[...]
    inv_ref[...] = invd


def _stage2(acc0, acc1, xr, w2l, w2r, b1, g1, be1, b2):
    row = lambda i: (i, 0)
    full = lambda i: (0, 0)
    return pl.pallas_call(
        _stage2_body,
        grid=(N_PAD // 256,),
        in_specs=[
            pl.BlockSpec((256, 32), row),
            pl.BlockSpec((256, 32), row),
            pl.BlockSpec((256, 16), row),
            pl.BlockSpec((16, 16), full),
            pl.BlockSpec((16, 16), full),
            pl.BlockSpec((1, 16), full),
            pl.BlockSpec((1, 16), full),
            pl.BlockSpec((1, 16), full),
            pl.BlockSpec((1, 16), full),
        ],
        out_specs=[
            pl.BlockSpec((256, 16), row),
            pl.BlockSpec((256, 16), row),
            pl.BlockSpec((256, 1), row),
        ],
        out_shape=[
            jax.ShapeDtypeStruct((N_PAD, 16), jnp.float32),
            jax.ShapeDtypeStruct((N_PAD, 16), jnp.float32),
            jax.ShapeDtypeStruct((N_PAD, 1), jnp.float32),
        ],
    )(acc0, acc1, xr, w2l, w2r, b1, g1, be1, b2)


def _stage3_body(a0_ref, a1_ref, r2_ref, inv_ref, batch_ref,
                 wf1_ref, bf1_ref, g2_ref, be2_ref, wf2_ref, bf2_ref,
                 out_ref, pool_ref):
    i = pl.program_id(0)
    nblk = pl.num_programs(0)
    h2n = jnp.maximum(
        (a0_ref[...] + a1_ref[...]) * inv_ref[...] + r2_ref[...],
        0.0)
    h2 = h2n.T
    b = batch_ref[...]

    @pl.when(i == 0)
    def _init():
        pool_ref[...] = jnp.full((16, 64), NEG_INF, jnp.float32)

    cols = []
    for g in range(64):
        cand = jnp.where(b == g, h2, NEG_INF)
        cols.append(jnp.max(cand, axis=1, keepdims=True))
    blk = jnp.concatenate(cols, axis=1)
    pool_ref[...] = jnp.maximum(pool_ref[...], blk)

    @pl.when(i == nblk - 1)
    def _head():
        pooled = pool_ref[...].T
        z = jnp.dot(pooled, wf1_ref[...],
                    preferred_element_type=jnp.float32) + bf1_ref[...]
        mu = jnp.mean(z, axis=1, keepdims=True)
        var = jnp.mean((z - mu) ** 2, axis=1, keepdims=True)
        z = (z - mu) * lax.rsqrt(var + 1e-5) * g2_ref[...] + be2_ref[...]
        z = jnp.maximum(z, 0.0)
        z = jnp.dot(z, wf2_ref[...],
                    preferred_element_type=jnp.float32) + bf2_ref[...]
        m = jnp.max(z, axis=1, keepdims=True)
        lse = m + jnp.log(jnp.sum(jnp.exp(z - m), axis=1, keepdims=True))
        out_ref[...] = z - lse


def _stage3(a0, a1, r2, inv, batch_t, wf1, bf1, g2, be2, wf2, bf2):
    row = lambda i: (i, 0)
    rowt = lambda i: (0, i)
    full = lambda i: (0, 0)
    out, _ = pl.pallas_call(
        _stage3_body,
        grid=(N_PAD // 1024,),
        in_specs=[
            pl.BlockSpec((1024, 16), row),
            pl.BlockSpec((1024, 16), row),
            pl.BlockSpec((1024, 16), row),
            pl.BlockSpec((1024, 1), row),
            pl.BlockSpec((1, 1024), rowt),
            pl.BlockSpec((16, 32), full),
            pl.BlockSpec((1, 32), full),
            pl.BlockSpec((1, 32), full),
            pl.BlockSpec((1, 32), full),
            pl.BlockSpec((32, 2), full),
            pl.BlockSpec((1, 2), full),
        ],
        out_specs=[
            pl.BlockSpec((64, 2), full),
            pl.BlockSpec((16, 64), full),
        ],
        out_shape=[
            jax.ShapeDtypeStruct((64, 2), jnp.float32),
            jax.ShapeDtypeStruct((16, 64), jnp.float32),
        ],
    )(a0, a1, r2, inv, batch_t, wf1, bf1, g2, be2, wf2, bf2)
    return out


def kernel(x, edge_index, edge_attr, batch, W1l, W1r, b1, g1, be1,
           W2l, W2r, b2, Wf1, bf1, g2, be2, Wf2, bf2):
    edge4 = jnp.full((2, E_PAD), N_NODES, jnp.int32).at[:, :E_EDGES].set(
        edge_index.astype(jnp.int32)).reshape(2, N_TILES, CHUNKS, CHUNK)
    batch_t = jnp.full((1, N_PAD), 64, jnp.int32).at[0, :N_NODES].set(
        batch.astype(jnp.int32))
    zero32 = jnp.zeros((N_PAD, 32), jnp.float32)
    zero16 = jnp.zeros((N_PAD, 16), jnp.float32)

    table1, xr = _stage1(x, W1l, W1r)
    acc1 = _sc_agg(table1, edge4, zero32, 32)
    table2, rest2, invdeg = _stage2(
        acc1[0], acc1[1], xr, W2l, W2r,
        b1.reshape(1, 16), g1.reshape(1, 16), be1.reshape(1, 16),
        b2.reshape(1, 16))
    acc2 = _sc_agg(table2, edge4, zero16, 16)
    return _stage3(acc2[0], acc2[1], rest2, invdeg, batch_t,
                   Wf1, bf1.reshape(1, 32), g2.reshape(1, 32),
                   be2.reshape(1, 32), Wf2, bf2.reshape(1, 2))


# speedup vs baseline: 1.0673x; 1.0673x over previous
import functools

import jax
import jax.numpy as jnp
from jax import lax
from jax.experimental import pallas as pl
from jax.experimental.pallas import tpu as pltpu
from jax.experimental.pallas import tpu_sc as plsc

N_NODES = 10000
N_PAD = 10240
E_EDGES = 160000
E_PAD = 163840
N_TILES = 32
CHUNKS = 40
CHUNK = 128
ROWS_PER_TILE = N_PAD // 16
NEG_INF = -3.0e38


def _stage1_body(x_ref, wl_ref, wr_ref, t1_ref, xr_ref):
    xb = x_ref[...]
    xl = jnp.dot(xb, wl_ref[...], preferred_element_type=jnp.float32)
    cols = lax.broadcasted_iota(jnp.int32, (400, 32), 1)
    t1_ref[...] = (jnp.concatenate([xl, jnp.zeros((400, 16), jnp.float32)],
                                   axis=1)
                   + jnp.where(cols == 16, 1.0, 0.0))
    xr_ref[...] = jnp.dot(xb, wr_ref[...], preferred_element_type=jnp.float32)


def _stage1(x, wl, wr):
    return pl.pallas_call(
        _stage1_body,
        grid=(N_NODES // 400,),
        in_specs=[
            pl.BlockSpec((400, 256), lambda i: (i, 0)),
            pl.BlockSpec((256, 16), lambda i: (0, 0)),
            pl.BlockSpec((256, 16), lambda i: (0, 0)),
        ],
        out_specs=[
            pl.BlockSpec((400, 32), lambda i: (i, 0)),
            pl.BlockSpec((400, 16), lambda i: (i, 0)),
        ],
        out_shape=[
            jax.ShapeDtypeStruct((N_PAD, 32), jnp.float32),
            jax.ShapeDtypeStruct((N_PAD, 16), jnp.float32),
        ],
    )(x, wl, wr)


@functools.lru_cache(maxsize=None)
def _make_sc_agg(width):
    mesh = plsc.VectorSubcoreMesh(core_axis_name="c", subcore_axis_name="s")

    @functools.partial(
        pl.kernel,
        mesh=mesh,
        out_type=jax.ShapeDtypeStruct((2, N_PAD, width), jnp.float32),
        scratch_types=[
            pltpu.VMEM((CHUNKS, CHUNK), jnp.int32),
            pltpu.VMEM((CHUNKS, CHUNK), jnp.int32),
            [pltpu.VMEM((CHUNK, width), jnp.float32) for _ in range(2)],
            pltpu.VMEM_SHARED((N_PAD, width), jnp.float32),
            [pltpu.SemaphoreType.DMA for _ in range(2)],
        ],
        compiler_params=pltpu.CompilerParams(use_tc_tiling_on_sc=False),
    )
    def sc_agg(table_hbm, src_hbm, dst_hbm, zero_hbm, out_hbm,
               src_v, dst_v, rows, acc_sh, gsem):
        c = lax.axis_index("c")
        s = lax.axis_index("s")
        wid = s * 2 + c
        row0 = s * ROWS_PER_TILE
        pltpu.sync_copy(zero_hbm.at[pl.ds(row0, ROWS_PER_TILE)],
                        acc_sh.at[pl.ds(row0, ROWS_PER_TILE)])
        pltpu.sync_copy(src_hbm.at[wid], src_v)
        pltpu.sync_copy(dst_hbm.at[wid], dst_v)
        plsc.subcore_barrier()

        for b in range(2):
            pltpu.async_copy(table_hbm.at[src_v.at[b]], rows[b], gsem[b])

        def body(i, carry):
            for b in range(2):
                j = 2 * i + b
                pltpu.make_async_copy(table_hbm.at[src_v.at[0]],
                                      rows[b], gsem[b]).wait()
                pltpu.sync_copy(rows[b], acc_sh.at[dst_v.at[j]], add=True)

                @pl.when(j + 2 < CHUNKS)
                def _prefetch():
                    pltpu.async_copy(table_hbm.at[src_v.at[j + 2]],
                                     rows[b], gsem[b])
            return carry

        lax.fori_loop(0, CHUNKS // 2, body, 0)
        plsc.subcore_barrier()
        pltpu.sync_copy(acc_sh.at[pl.ds(row0, ROWS_PER_TILE)],
                        out_hbm.at[c, pl.ds(row0, ROWS_PER_TILE)])

    return sc_agg


def _sc_agg(table, src3, dst3, zero, width):
    return _make_sc_agg(width)(table, src3, dst3, zero)


def _stage2_body(a0_ref, a1_ref, xr_ref, w2l_ref, w2r_ref, b1_ref, g1_ref,
                 be1_ref, b2_ref, t2_ref, r2_ref, inv_ref):
    i = pl.program_id(0)
    acc = a0_ref[...] + a1_ref[...]
    agg = acc[:, :16]
    deg = acc[:, 16:17]
    invd = 1.0 / jnp.maximum(deg, 1.0)
    pre = agg * invd + b1_ref[...] + xr_ref[...]
    h = jnp.maximum(pre, 0.0)
    mu = jnp.mean(h, axis=1, keepdims=True)
    var = jnp.mean((h - mu) ** 2, axis=1, keepdims=True)
    hn = (h - mu) * lax.rsqrt(var + 1e-5) * g1_ref[...] + be1_ref[...]
    rows = i * 256 + lax.broadcasted_iota(jnp.int32, (256, 1), 0)
    valid = rows < N_NODES
    hl = jnp.dot(hn, w2l_ref[...], preferred_element_type=jnp.float32)
    t2_ref[...] = jnp.where(valid, hl, 0.0)
    r2_ref[...] = jnp.dot(hn, w2r_ref[...],
                          preferred_element_type=jnp.float32) + b2_ref[...]
    inv_ref[...] = invd


def _stage2(acc0, acc1, xr, w2l, w2r, b1, g1, be1, b2):
    row = lambda i: (i, 0)
    full = lambda i: (0, 0)
    return pl.pallas_call(
        _stage2_body,
        grid=(N_PAD // 256,),
        in_specs=[
            pl.BlockSpec((256, 32), row),
            pl.BlockSpec((256, 32), row),
            pl.BlockSpec((256, 16), row),
            pl.BlockSpec((16, 16), full),
            pl.BlockSpec((16, 16), full),
            pl.BlockSpec((1, 16), full),
            pl.BlockSpec((1, 16), full),
            pl.BlockSpec((1, 16), full),
            pl.BlockSpec((1, 16), full),
        ],
        out_specs=[
            pl.BlockSpec((256, 16), row),
            pl.BlockSpec((256, 16), row),
            pl.BlockSpec((256, 1), row),
        ],
        out_shape=[
            jax.ShapeDtypeStruct((N_PAD, 16), jnp.float32),
            jax.ShapeDtypeStruct((N_PAD, 16), jnp.float32),
            jax.ShapeDtypeStruct((N_PAD, 1), jnp.float32),
        ],
    )(acc0, acc1, xr, w2l, w2r, b1, g1, be1, b2)


def _stage3_body(a0_ref, a1_ref, r2_ref, inv_ref, batch_ref,
                 wf1_ref, bf1_ref, g2_ref, be2_ref, wf2_ref, bf2_ref,
                 out_ref, pool_ref):
    i = pl.program_id(0)
    nblk = pl.num_programs(0)
    h2n = jnp.maximum(
        (a0_ref[...] + a1_ref[...]) * inv_ref[...] + r2_ref[...],
        0.0)
    h2 = h2n.T
    b = batch_ref[...]

    @pl.when(i == 0)
    def _init():
        pool_ref[...] = jnp.full((16, 64), NEG_INF, jnp.float32)

    cols = []
    for g in range(64):
        cand = jnp.where(b == g, h2, NEG_INF)
        cols.append(jnp.max(cand, axis=1, keepdims=True))
    blk = jnp.concatenate(cols, axis=1)
    pool_ref[...] = jnp.maximum(pool_ref[...], blk)

    @pl.when(i == nblk - 1)
    def _head():
        pooled = pool_ref[...].T
        z = jnp.dot(pooled, wf1_ref[...],
                    preferred_element_type=jnp.float32) + bf1_ref[...]
        mu = jnp.mean(z, axis=1, keepdims=True)
        var = jnp.mean((z - mu) ** 2, axis=1, keepdims=True)
        z = (z - mu) * lax.rsqrt(var + 1e-5) * g2_ref[...] + be2_ref[...]
        z = jnp.maximum(z, 0.0)
        z = jnp.dot(z, wf2_ref[...],
                    preferred_element_type=jnp.float32) + bf2_ref[...]
        m = jnp.max(z, axis=1, keepdims=True)
        lse = m + jnp.log(jnp.sum(jnp.exp(z - m), axis=1, keepdims=True))
        out_ref[...] = z - lse


def _stage3(a0, a1, r2, inv, batch_t, wf1, bf1, g2, be2, wf2, bf2):
    row = lambda i: (i, 0)
    rowt = lambda i: (0, i)
    full = lambda i: (0, 0)
    out, _ = pl.pallas_call(
        _stage3_body,
        grid=(N_PAD // 1024,),
        in_specs=[
            pl.BlockSpec((1024, 16), row),
            pl.BlockSpec((1024, 16), row),
            pl.BlockSpec((1024, 16), row),
            pl.BlockSpec((1024, 1), row),
            pl.BlockSpec((1, 1024), rowt),
            pl.BlockSpec((16, 32), full),
            pl.BlockSpec((1, 32), full),
            pl.BlockSpec((1, 32), full),
            pl.BlockSpec((1, 32), full),
            pl.BlockSpec((32, 2), full),
            pl.BlockSpec((1, 2), full),
        ],
        out_specs=[
            pl.BlockSpec((64, 2), full),
            pl.BlockSpec((16, 64), full),
        ],
        out_shape=[
            jax.ShapeDtypeStruct((64, 2), jnp.float32),
            jax.ShapeDtypeStruct((16, 64), jnp.float32),
        ],
    )(a0, a1, r2, inv, batch_t, wf1, bf1, g2, be2, wf2, bf2)
    return out


def kernel(x, edge_index, edge_attr, batch, W1l, W1r, b1, g1, be1,
           W2l, W2r, b2, Wf1, bf1, g2, be2, Wf2, bf2):
    src3 = jnp.full((E_PAD,), N_NODES, jnp.int32).at[:E_EDGES].set(
        edge_index[0].astype(jnp.int32)).reshape(N_TILES, CHUNKS, CHUNK)
    dst3 = jnp.full((E_PAD,), N_NODES, jnp.int32).at[:E_EDGES].set(
        edge_index[1].astype(jnp.int32)).reshape(N_TILES, CHUNKS, CHUNK)
    batch_t = jnp.full((1, N_PAD), 64, jnp.int32).at[0, :N_NODES].set(
        batch.astype(jnp.int32))
    zero32 = jnp.zeros((N_PAD, 32), jnp.float32)
    zero16 = jnp.zeros((N_PAD, 16), jnp.float32)

    table1, xr = _stage1(x, W1l, W1r)
    acc1 = _sc_agg(table1, src3, dst3, zero32, 32)
    table2, rest2, invdeg = _stage2(
        acc1[0], acc1[1], xr, W2l, W2r,
        b1.reshape(1, 16), g1.reshape(1, 16), be1.reshape(1, 16),
        b2.reshape(1, 16))
    acc2 = _sc_agg(table2, src3, dst3, zero16, 16)
    return _stage3(acc2[0], acc2[1], rest2, invdeg, batch_t,
                   Wf1, bf1.reshape(1, 32), g2.reshape(1, 32),
                   be2.reshape(1, 32), Wf2, bf2.reshape(1, 2))


# speedup vs baseline: 1.1167x; 1.0463x over previous
import functools

import jax
import jax.numpy as jnp
from jax import lax
from jax.experimental import pallas as pl
from jax.experimental.pallas import tpu as pltpu
from jax.experimental.pallas import tpu_sc as plsc

N_NODES = 10000
N_PAD = 10240
E_EDGES = 160000
E_PAD = 163840
N_TILES = 32
CHUNKS = 40
CHUNK = 128
ROWS_PER_TILE = N_PAD // 16
NEG_INF = -3.0e38


def _stage1_body(x_ref, wl_ref, wr_ref, t1_ref, xr_ref):
    xb = x_ref[...]
    t1_ref[...] = jnp.dot(xb, wl_ref[...], preferred_element_type=jnp.float32)
    xr_ref[...] = jnp.dot(xb, wr_ref[...], preferred_element_type=jnp.float32)


def _stage1(x, wl, wr):
    return pl.pallas_call(
        _stage1_body,
        grid=(N_NODES // 400,),
        in_specs=[
            pl.BlockSpec((400, 256), lambda i: (i, 0)),
            pl.BlockSpec((256, 16), lambda i: (0, 0)),
            pl.BlockSpec((256, 16), lambda i: (0, 0)),
        ],
        out_specs=[
            pl.BlockSpec((400, 16), lambda i: (i, 0)),
            pl.BlockSpec((400, 16), lambda i: (i, 0)),
        ],
        out_shape=[
            jax.ShapeDtypeStruct((N_PAD, 16), jnp.float32),
            jax.ShapeDtypeStruct((N_PAD, 16), jnp.float32),
        ],
    )(x, wl, wr)


@functools.lru_cache(maxsize=None)
def _make_sc_agg(width):
    mesh = plsc.VectorSubcoreMesh(core_axis_name="c", subcore_axis_name="s")

    @functools.partial(
        pl.kernel,
        mesh=mesh,
        out_type=jax.ShapeDtypeStruct((2, N_PAD, width), jnp.float32),
        scratch_types=[
            pltpu.VMEM((CHUNKS, CHUNK), jnp.int32),
            pltpu.VMEM((CHUNKS, CHUNK), jnp.int32),
            [pltpu.VMEM((CHUNK, width), jnp.float32) for _ in range(2)],
            pltpu.VMEM_SHARED((N_PAD, width), jnp.float32),
            [pltpu.SemaphoreType.DMA for _ in range(2)],
        ],
        compiler_params=pltpu.CompilerParams(use_tc_tiling_on_sc=False),
    )
    def sc_agg(table_hbm, src_hbm, dst_hbm, zero_hbm, out_hbm,
               src_v, dst_v, rows, acc_sh, gsem):
        c = lax.axis_index("c")
        s = lax.axis_index("s")
        wid = s * 2 + c
        row0 = s * ROWS_PER_TILE
        pltpu.sync_copy(zero_hbm.at[pl.ds(row0, ROWS_PER_TILE)],
                        acc_sh.at[pl.ds(row0, ROWS_PER_TILE)])
        pltpu.sync_copy(src_hbm.at[wid], src_v)
        pltpu.sync_copy(dst_hbm.at[wid], dst_v)
        plsc.subcore_barrier()

        for b in range(2):
            pltpu.async_copy(table_hbm.at[src_v.at[b]], rows[b], gsem[b])

        def body(i, carry):
            for b in range(2):
                j = 2 * i + b
                pltpu.make_async_copy(table_hbm.at[src_v.at[0]],
                                      rows[b], gsem[b]).wait()
                pltpu.sync_copy(rows[b], acc_sh.at[dst_v.at[j]], add=True)

                @pl.when(j + 2 < CHUNKS)
                def _prefetch():
                    pltpu.async_copy(table_hbm.at[src_v.at[j + 2]],
                                     rows[b], gsem[b])
            return carry

        lax.fori_loop(0, CHUNKS // 2, body, 0)
        plsc.subcore_barrier()
        pltpu.sync_copy(acc_sh.at[pl.ds(row0, ROWS_PER_TILE)],
                        out_hbm.at[c, pl.ds(row0, ROWS_PER_TILE)])

    return sc_agg


def _sc_agg(table, src3, dst3, zero, width):
    return _make_sc_agg(width)(table, src3, dst3, zero)


@functools.lru_cache(maxsize=None)
def _make_sc_agg_deg():
    mesh = plsc.VectorSubcoreMesh(core_axis_name="c", subcore_axis_name="s")
    width = 16

    @functools.partial(
        pl.kernel,
        mesh=mesh,
        out_type=[
            jax.ShapeDtypeStruct((2, N_PAD, width), jnp.float32),
            jax.ShapeDtypeStruct((2, N_PAD), jnp.float32),
        ],
        scratch_types=[
            pltpu.VMEM((CHUNKS, CHUNK), jnp.int32),
            pltpu.VMEM((CHUNKS, CHUNK), jnp.int32),
            [pltpu.VMEM((CHUNK, width), jnp.float32) for _ in range(2)],
            pltpu.VMEM((N_PAD,), jnp.float32),
            pltpu.VMEM((16, ROWS_PER_TILE), jnp.float32),
            pltpu.VMEM((ROWS_PER_TILE,), jnp.float32),
            pltpu.VMEM_SHARED((N_PAD, width), jnp.float32),
            pltpu.VMEM_SHARED((16, N_PAD), jnp.float32),
            [pltpu.SemaphoreType.DMA for _ in range(2)],
            [pltpu.SemaphoreType.DMA for _ in range(2)],
        ],
        compiler_params=pltpu.CompilerParams(use_tc_tiling_on_sc=False,
                                             needs_layout_passes=False),
    )
    def sc_agg_deg(table_hbm, src_hbm, dst_hbm, zero_hbm, out_hbm, deg_hbm,
                   src_v, dst_v, rows, deg_v, red_v, dsum_v, acc_sh, deg_sh,
                   gsem, ssem):
        c = lax.axis_index("c")
        s = lax.axis_index("s")
        wid = s * 2 + c
        row0 = s * ROWS_PER_TILE
        pltpu.sync_copy(zero_hbm.at[pl.ds(row0, ROWS_PER_TILE)],
                        acc_sh.at[pl.ds(row0, ROWS_PER_TILE)])
        pltpu.sync_copy(src_hbm.at[wid], src_v)
        pltpu.sync_copy(dst_hbm.at[wid], dst_v)
        zeros16 = jnp.zeros((16,), jnp.float32)
        ones16 = jnp.ones((16,), jnp.float32)

        def zbody(i, carry):
            deg_v[pl.ds(i * 16, 16)] = zeros16
            return carry

        lax.fori_loop(0, N_PAD // 16, zbody, 0)
        plsc.subcore_barrier()

        for b in range(2):
            pltpu.async_copy(table_hbm.at[src_v.at[b]], rows[b], gsem[b])

        def body(i, carry):
            for b in range(2):
                j = 2 * i + b
                pltpu.make_async_copy(table_hbm.at[src_v.at[0]],
                                      rows[b], gsem[b]).wait()
                pltpu.async_copy(rows[b], acc_sh.at[dst_v.at[j]],
                                 ssem[b], add=True)
                for k in range(8):
                    idx16 = dst_v[j, pl.ds(k * 16, 16)]
                    plsc.addupdate_scatter(deg_v, [idx16], ones16)

                @pl.when(j + 2 < CHUNKS)
                def _prefetch():
                    pltpu.make_async_copy(rows[b], acc_sh.at[dst_v.at[0]],
                                          ssem[b]).wait()
                    pltpu.async_copy(table_hbm.at[src_v.at[j + 2]],
                                     rows[b], gsem[b])
            return carry

        lax.fori_loop(0, CHUNKS // 2, body, 0)
        for b in range(2):
            pltpu.make_async_copy(rows[b], acc_sh.at[dst_v.at[0]],
                                  ssem[b]).wait()
        pltpu.sync_copy(deg_v, deg_sh.at[s])
        plsc.subcore_barrier()
        pltpu.sync_copy(acc_sh.at[pl.ds(row0, ROWS_PER_TILE)],
                        out_hbm.at[c, pl.ds(row0, ROWS_PER_TILE)])
        pltpu.sync_copy(deg_sh.at[pl.ds(0, 16), pl.ds(row0, ROWS_PER_TILE)],
                        red_v)

        def rbody(i, carry):
            t = red_v[0, pl.ds(i * 16, 16)]
            for k in range(1, 16):
                t = t + red_v[k, pl.ds(i * 16, 16)]
            dsum_v[pl.ds(i * 16, 16)] = t
            return carry

        lax.fori_loop(0, ROWS_PER_TILE // 16, rbody, 0)
        pltpu.sync_copy(dsum_v, deg_hbm.at[c, pl.ds(row0, ROWS_PER_TILE)])

    return sc_agg_deg


def _stage2_body(a0_ref, a1_ref, deg_ref, xr_ref, w2l_ref, w2r_ref, b1_ref,
                 g1_ref, be1_ref, b2_ref, t2_ref, r2_ref, inv_ref):
    i = pl.program_id(0)
    agg = a0_ref[...] + a1_ref[...]
    deg = deg_ref[...]
    invd = 1.0 / jnp.maximum(deg, 1.0)
    pre = agg * invd + b1_ref[...] + xr_ref[...]
    h = jnp.maximum(pre, 0.0)
    mu = jnp.mean(h, axis=1, keepdims=True)
    var = jnp.mean((h - mu) ** 2, axis=1, keepdims=True)
    hn = (h - mu) * lax.rsqrt(var + 1e-5) * g1_ref[...] + be1_ref[...]
    rows = i * 256 + lax.broadcasted_iota(jnp.int32, (256, 1), 0)
    valid = rows < N_NODES
    hl = jnp.dot(hn, w2l_ref[...], preferred_element_type=jnp.float32)
    t2_ref[...] = jnp.where(valid, hl, 0.0)
    r2_ref[...] = jnp.dot(hn, w2r_ref[...],
                          preferred_element_type=jnp.float32) + b2_ref[...]
    inv_ref[...] = invd


def _stage2(acc0, acc1, deg, xr, w2l, w2r, b1, g1, be1, b2):
    row = lambda i: (i, 0)
    full = lambda i: (0, 0)
    return pl.pallas_call(
        _stage2_body,
        grid=(N_PAD // 256,),
        in_specs=[
            pl.BlockSpec((256, 16), row),
            pl.BlockSpec((256, 16), row),
            pl.BlockSpec((256, 1), row),
            pl.BlockSpec((256, 16), row),
            pl.BlockSpec((16, 16), full),
            pl.BlockSpec((16, 16), full),
            pl.BlockSpec((1, 16), full),
            pl.BlockSpec((1, 16), full),
            pl.BlockSpec((1, 16), full),
            pl.BlockSpec((1, 16), full),
        ],
        out_specs=[
            pl.BlockSpec((256, 16), row),
            pl.BlockSpec((256, 16), row),
            pl.BlockSpec((256, 1), row),
        ],
        out_shape=[
            jax.ShapeDtypeStruct((N_PAD, 16), jnp.float32),
            jax.ShapeDtypeStruct((N_PAD, 16), jnp.float32),
            jax.ShapeDtypeStruct((N_PAD, 1), jnp.float32),
        ],
    )(acc0, acc1, deg, xr, w2l, w2r, b1, g1, be1, b2)


def _stage3_body(a0_ref, a1_ref, r2_ref, inv_ref, batch_ref,
                 wf1_ref, bf1_ref, g2_ref, be2_ref, wf2_ref, bf2_ref,
                 out_ref, pool_ref):
    i = pl.program_id(0)
    nblk = pl.num_programs(0)
    h2n = jnp.maximum(
        (a0_ref[...] + a1_ref[...]) * inv_ref[...] + r2_ref[...],
        0.0)
    h2 = h2n.T
    b = batch_ref[...]

    @pl.when(i == 0)
    def _init():
        pool_ref[...] = jnp.full((16, 64), NEG_INF, jnp.float32)

    cols = []
    for g in range(64):
        cand = jnp.where(b == g, h2, NEG_INF)
        cols.append(jnp.max(cand, axis=1, keepdims=True))
    blk = jnp.concatenate(cols, axis=1)
    pool_ref[...] = jnp.maximum(pool_ref[...], blk)

    @pl.when(i == nblk - 1)
    def _head():
        pooled = pool_ref[...].T
        z = jnp.dot(pooled, wf1_ref[...],
                    preferred_element_type=jnp.float32) + bf1_ref[...]
        mu = jnp.mean(z, axis=1, keepdims=True)
        var = jnp.mean((z - mu) ** 2, axis=1, keepdims=True)
        z = (z - mu) * lax.rsqrt(var + 1e-5) * g2_ref[...] + be2_ref[...]
        z = jnp.maximum(z, 0.0)
        z = jnp.dot(z, wf2_ref[...],
                    preferred_element_type=jnp.float32) + bf2_ref[...]
        m = jnp.max(z, axis=1, keepdims=True)
        lse = m + jnp.log(jnp.sum(jnp.exp(z - m), axis=1, keepdims=True))
        out_ref[...] = z - lse


def _stage3(a0, a1, r2, inv, batch_t, wf1, bf1, g2, be2, wf2, bf2):
    row = lambda i: (i, 0)
    rowt = lambda i: (0, i)
    full = lambda i: (0, 0)
    out, _ = pl.pallas_call(
        _stage3_body,
        grid=(N_PAD // 1024,),
        in_specs=[
            pl.BlockSpec((1024, 16), row),
            pl.BlockSpec((1024, 16), row),
            pl.BlockSpec((1024, 16), row),
            pl.BlockSpec((1024, 1), row),
            pl.BlockSpec((1, 1024), rowt),
            pl.BlockSpec((16, 32), full),
            pl.BlockSpec((1, 32), full),
            pl.BlockSpec((1, 32), full),
            pl.BlockSpec((1, 32), full),
            pl.BlockSpec((32, 2), full),
            pl.BlockSpec((1, 2), full),
        ],
        out_specs=[
            pl.BlockSpec((64, 2), full),
            pl.BlockSpec((16, 64), full),
        ],
        out_shape=[
            jax.ShapeDtypeStruct((64, 2), jnp.float32),
            jax.ShapeDtypeStruct((16, 64), jnp.float32),
        ],
    )(a0, a1, r2, inv, batch_t, wf1, bf1, g2, be2, wf2, bf2)
    return out


def kernel(x, edge_index, edge_attr, batch, W1l, W1r, b1, g1, be1,
           W2l, W2r, b2, Wf1, bf1, g2, be2, Wf2, bf2):
    src3 = jnp.full((E_PAD,), N_NODES, jnp.int32).at[:E_EDGES].set(
        edge_index[0].astype(jnp.int32)).reshape(N_TILES, CHUNKS, CHUNK)
    dst3 = jnp.full((E_PAD,), N_NODES, jnp.int32).at[:E_EDGES].set(
        edge_index[1].astype(jnp.int32)).reshape(N_TILES, CHUNKS, CHUNK)
    batch_t = jnp.full((1, N_PAD), 64, jnp.int32).at[0, :N_NODES].set(
        batch.astype(jnp.int32))
    zero16 = jnp.zeros((N_PAD, 16), jnp.float32)

    table1, xr = _stage1(x, W1l, W1r)
    acc1, degp = _make_sc_agg_deg()(table1, src3, dst3, zero16)
    deg = (degp[0] + degp[1]).reshape(N_PAD, 1)
    table2, rest2, invdeg = _stage2(
        acc1[0], acc1[1], deg, xr, W2l, W2r,
        b1.reshape(1, 16), g1.reshape(1, 16), be1.reshape(1, 16),
        b2.reshape(1, 16))
    acc2 = _sc_agg(table2, src3, dst3, zero16, 16)
    return _stage3(acc2[0], acc2[1], rest2, invdeg, batch_t,
                   Wf1, bf1.reshape(1, 32), g2.reshape(1, 32),
                   be2.reshape(1, 32), Wf2, bf2.reshape(1, 2))


# speedup vs baseline: 1.2100x; 1.0836x over previous
import functools

import jax
import jax.numpy as jnp
from jax import lax
from jax.experimental import pallas as pl
from jax.experimental.pallas import tpu as pltpu
from jax.experimental.pallas import tpu_sc as plsc

N_NODES = 10000
N_PAD = 10240
E_EDGES = 160000
E_PAD = 163840
N_TILES = 32
CHUNKS = 40
CHUNK = 128
ROWS_PER_TILE = N_PAD // 16
NEG_INF = -3.0e38


def _stage1_body(x_ref, wl_ref, wr_ref, t1_ref, xr_ref):
    xb = x_ref[...]
    t1_ref[...] = jnp.dot(xb, wl_ref[...], preferred_element_type=jnp.float32)
    xr_ref[...] = jnp.dot(xb, wr_ref[...], preferred_element_type=jnp.float32)


def _stage1(x, wl, wr):
    return pl.pallas_call(
        _stage1_body,
        grid=(N_NODES // 400,),
        in_specs=[
            pl.BlockSpec((400, 256), lambda i: (i, 0)),
            pl.BlockSpec((256, 16), lambda i: (0, 0)),
            pl.BlockSpec((256, 16), lambda i: (0, 0)),
        ],
        out_specs=[
            pl.BlockSpec((400, 16), lambda i: (i, 0)),
            pl.BlockSpec((400, 16), lambda i: (i, 0)),
        ],
        out_shape=[
            jax.ShapeDtypeStruct((N_PAD, 16), jnp.float32),
            jax.ShapeDtypeStruct((N_PAD, 16), jnp.float32),
        ],
    )(x, wl, wr)


@functools.lru_cache(maxsize=None)
def _make_sc_agg(width):
    mesh = plsc.VectorSubcoreMesh(core_axis_name="c", subcore_axis_name="s")

    @functools.partial(
        pl.kernel,
        mesh=mesh,
        out_type=jax.ShapeDtypeStruct((2, N_PAD, width), jnp.float32),
        scratch_types=[
            pltpu.VMEM((2, CHUNKS, CHUNK), jnp.int32),
            [pltpu.VMEM((CHUNK, width), jnp.float32) for _ in range(2)],
            pltpu.VMEM_SHARED((N_PAD, width), jnp.float32),
            [pltpu.SemaphoreType.DMA for _ in range(2)],
        ],
        compiler_params=pltpu.CompilerParams(use_tc_tiling_on_sc=False),
    )
    def sc_agg(table_hbm, edge_hbm, zero_hbm, out_hbm,
               idx_v, rows, acc_sh, gsem):
        c = lax.axis_index("c")
        s = lax.axis_index("s")
        wid = s * 2 + c
        row0 = s * ROWS_PER_TILE
        pltpu.sync_copy(zero_hbm.at[pl.ds(row0, ROWS_PER_TILE)],
                        acc_sh.at[pl.ds(row0, ROWS_PER_TILE)])
        pltpu.sync_copy(edge_hbm.at[wid], idx_v)
        plsc.subcore_barrier()
        src_v = idx_v.at[0]
        dst_v = idx_v.at[1]

        for b in range(2):
            pltpu.async_copy(table_hbm.at[src_v.at[b]], rows[b], gsem[b])

        def body(i, carry):
            for b in range(2):
                j = 2 * i + b
                pltpu.make_async_copy(table_hbm.at[src_v.at[0]],
                                      rows[b], gsem[b]).wait()
                pltpu.sync_copy(rows[b], acc_sh.at[dst_v.at[j]], add=True)

                @pl.when(j + 2 < CHUNKS)
                def _prefetch():
                    pltpu.async_copy(table_hbm.at[src_v.at[j + 2]],
                                     rows[b], gsem[b])
            return carry

        lax.fori_loop(0, CHUNKS // 2, body, 0)
        plsc.subcore_barrier()
        pltpu.sync_copy(acc_sh.at[pl.ds(row0, ROWS_PER_TILE)],
                        out_hbm.at[c, pl.ds(row0, ROWS_PER_TILE)])

    return sc_agg


def _sc_agg(table, edge4, zero, width):
    return _make_sc_agg(width)(table, edge4, zero)


@functools.lru_cache(maxsize=None)
def _make_sc_agg_deg():
    mesh = plsc.VectorSubcoreMesh(core_axis_name="c", subcore_axis_name="s")
    width = 16

    @functools.partial(
        pl.kernel,
        mesh=mesh,
        out_type=[
            jax.ShapeDtypeStruct((2, N_PAD, width), jnp.float32),
            jax.ShapeDtypeStruct((2, N_PAD), jnp.float32),
        ],
        scratch_types=[
            pltpu.VMEM((2, CHUNKS, CHUNK), jnp.int32),
            [pltpu.VMEM((CHUNK, width), jnp.float32) for _ in range(2)],
            pltpu.VMEM((N_PAD,), jnp.float32),
            pltpu.VMEM((16, ROWS_PER_TILE), jnp.float32),
            pltpu.VMEM((ROWS_PER_TILE,), jnp.float32),
            pltpu.VMEM_SHARED((N_PAD, width), jnp.float32),
            pltpu.VMEM_SHARED((16, N_PAD), jnp.float32),
            [pltpu.SemaphoreType.DMA for _ in range(2)],
            [pltpu.SemaphoreType.DMA for _ in range(2)],
        ],
        compiler_params=pltpu.CompilerParams(use_tc_tiling_on_sc=False,
                                             needs_layout_passes=False),
    )
    def sc_agg_deg(table_hbm, edge_hbm, zero_hbm, out_hbm, deg_hbm,
                   idx_v, rows, deg_v, red_v, dsum_v, acc_sh, deg_sh,
                   gsem, ssem):
        c = lax.axis_index("c")
        s = lax.axis_index("s")
        wid = s * 2 + c
        row0 = s * ROWS_PER_TILE
        pltpu.sync_copy(zero_hbm.at[pl.ds(row0, ROWS_PER_TILE)],
                        acc_sh.at[pl.ds(row0, ROWS_PER_TILE)])
        pltpu.sync_copy(edge_hbm.at[wid], idx_v)
        src_v = idx_v.at[0]
        dst_v = idx_v.at[1]
        zeros16 = jnp.zeros((16,), jnp.float32)
        ones16 = jnp.ones((16,), jnp.float32)

        def zbody(i, carry):
            deg_v[pl.ds(i * 16, 16)] = zeros16
            return carry

        lax.fori_loop(0, N_PAD // 16, zbody, 0)
        plsc.subcore_barrier()

        for b in range(2):
            pltpu.async_copy(table_hbm.at[src_v.at[b]], rows[b], gsem[b])

        def body(i, carry):
            for b in range(2):
                j = 2 * i + b
                pltpu.make_async_copy(table_hbm.at[src_v.at[0]],
                                      rows[b], gsem[b]).wait()
                pltpu.async_copy(rows[b], acc_sh.at[dst_v.at[j]],
                                 ssem[b], add=True)
                for k in range(8):
                    idx16 = idx_v[1, j, pl.ds(k * 16, 16)]
                    plsc.addupdate_scatter(deg_v, [idx16], ones16)

                @pl.when(j + 2 < CHUNKS)
                def _prefetch():
                    pltpu.make_async_copy(rows[b], acc_sh.at[dst_v.at[0]],
                                          ssem[b]).wait()
                    pltpu.async_copy(table_hbm.at[src_v.at[j + 2]],
                                     rows[b], gsem[b])
            return carry

        lax.fori_loop(0, CHUNKS // 2, body, 0)
        for b in range(2):
            pltpu.make_async_copy(rows[b], acc_sh.at[dst_v.at[0]],
                                  ssem[b]).wait()
        pltpu.sync_copy(deg_v, deg_sh.at[s])
        plsc.subcore_barrier()
        pltpu.sync_copy(acc_sh.at[pl.ds(row0, ROWS_PER_TILE)],
                        out_hbm.at[c, pl.ds(row0, ROWS_PER_TILE)])
        pltpu.sync_copy(deg_sh.at[pl.ds(0, 16), pl.ds(row0, ROWS_PER_TILE)],
                        red_v)

        def rbody(i, carry):
            t = red_v[0, pl.ds(i * 16, 16)]
            for k in range(1, 16):
                t = t + red_v[k, pl.ds(i * 16, 16)]
            dsum_v[pl.ds(i * 16, 16)] = t
            return carry

        lax.fori_loop(0, ROWS_PER_TILE // 16, rbody, 0)
        pltpu.sync_copy(dsum_v, deg_hbm.at[c, pl.ds(row0, ROWS_PER_TILE)])

    return sc_agg_deg


def _stage2_body(a0_ref, a1_ref, deg_ref, xr_ref, w2l_ref, w2r_ref, b1_ref,
                 g1_ref, be1_ref, b2_ref, t2_ref, r2_ref, inv_ref):
    i = pl.program_id(0)
    agg = a0_ref[...] + a1_ref[...]
    dd = deg_ref[...].T
    deg = dd[:, 0:1] + dd[:, 1:2]
    invd = 1.0 / jnp.maximum(deg, 1.0)
    pre = agg * invd + b1_ref[...] + xr_ref[...]
    h = jnp.maximum(pre, 0.0)
    mu = jnp.mean(h, axis=1, keepdims=True)
    var = jnp.mean((h - mu) ** 2, axis=1, keepdims=True)
    hn = (h - mu) * lax.rsqrt(var + 1e-5) * g1_ref[...] + be1_ref[...]
    rows = i * 256 + lax.broadcasted_iota(jnp.int32, (256, 1), 0)
    valid = rows < N_NODES
    hl = jnp.dot(hn, w2l_ref[...], preferred_element_type=jnp.float32)
    t2_ref[...] = jnp.where(valid, hl, 0.0)
    r2_ref[...] = jnp.dot(hn, w2r_ref[...],
                          preferred_element_type=jnp.float32) + b2_ref[...]
    inv_ref[...] = invd


def _stage2(acc0, acc1, deg, xr, w2l, w2r, b1, g1, be1, b2):
    row = lambda i: (i, 0)
    full = lambda i: (0, 0)
    return pl.pallas_call(
        _stage2_body,
        grid=(N_PAD // 256,),
        in_specs=[
            pl.BlockSpec((256, 16), row),
            pl.BlockSpec((256, 16), row),
            pl.BlockSpec((2, 256), lambda i: (0, i)),
            pl.BlockSpec((256, 16), row),
            pl.BlockSpec((16, 16), full),
            pl.BlockSpec((16, 16), full),
            pl.BlockSpec((1, 16), full),
            pl.BlockSpec((1, 16), full),
            pl.BlockSpec((1, 16), full),
            pl.BlockSpec((1, 16), full),
        ],
        out_specs=[
            pl.BlockSpec((256, 16), row),
            pl.BlockSpec((256, 16), row),
            pl.BlockSpec((256, 1), row),
        ],
        out_shape=[
            jax.ShapeDtypeStruct((N_PAD, 16), jnp.float32),
            jax.ShapeDtypeStruct((N_PAD, 16), jnp.float32),
            jax.ShapeDtypeStruct((N_PAD, 1), jnp.float32),
        ],
    )(acc0, acc1, deg, xr, w2l, w2r, b1, g1, be1, b2)


def _stage3_body(a0_ref, a1_ref, r2_ref, inv_ref, batch_ref,
                 wf1_ref, bf1_ref, g2_ref, be2_ref, wf2_ref, bf2_ref,
                 out_ref, pool_ref):
    i = pl.program_id(0)
    nblk = pl.num_programs(0)
    h2n = jnp.maximum(
        (a0_ref[...] + a1_ref[...]) * inv_ref[...] + r2_ref[...],
        0.0)
    h2 = h2n.T
    b = batch_ref[...]

    @pl.when(i == 0)
    def _init():
        pool_ref[...] = jnp.full((16, 64), NEG_INF, jnp.float32)

    cols = []
    for g in range(64):
        cand = jnp.where(b == g, h2, NEG_INF)
        cols.append(jnp.max(cand, axis=1, keepdims=True))
    blk = jnp.concatenate(cols, axis=1)
    pool_ref[...] = jnp.maximum(pool_ref[...], blk)

    @pl.when(i == nblk - 1)
    def _head():
        pooled = pool_ref[...].T
        z = jnp.dot(pooled, wf1_ref[...],
                    preferred_element_type=jnp.float32) + bf1_ref[...]
        mu = jnp.mean(z, axis=1, keepdims=True)
        var = jnp.mean((z - mu) ** 2, axis=1, keepdims=True)
        z = (z - mu) * lax.rsqrt(var + 1e-5) * g2_ref[...] + be2_ref[...]
        z = jnp.maximum(z, 0.0)
        z = jnp.dot(z, wf2_ref[...],
                    preferred_element_type=jnp.float32) + bf2_ref[...]
        m = jnp.max(z, axis=1, keepdims=True)
        lse = m + jnp.log(jnp.sum(jnp.exp(z - m), axis=1, keepdims=True))
        out_ref[...] = z - lse


def _stage3(a0, a1, r2, inv, batch_t, wf1, bf1, g2, be2, wf2, bf2):
    row = lambda i: (i, 0)
    rowt = lambda i: (0, i)
    full = lambda i: (0, 0)
    out, _ = pl.pallas_call(
        _stage3_body,
        grid=(N_PAD // 1024,),
        in_specs=[
            pl.BlockSpec((1024, 16), row),
            pl.BlockSpec((1024, 16), row),
            pl.BlockSpec((1024, 16), row),
            pl.BlockSpec((1024, 1), row),
            pl.BlockSpec((1, 1024), rowt),
            pl.BlockSpec((16, 32), full),
            pl.BlockSpec((1, 32), full),
            pl.BlockSpec((1, 32), full),
            pl.BlockSpec((1, 32), full),
            pl.BlockSpec((32, 2), full),
            pl.BlockSpec((1, 2), full),
        ],
        out_specs=[
            pl.BlockSpec((64, 2), full),
            pl.BlockSpec((16, 64), full),
        ],
        out_shape=[
            jax.ShapeDtypeStruct((64, 2), jnp.float32),
            jax.ShapeDtypeStruct((16, 64), jnp.float32),
        ],
    )(a0, a1, r2, inv, batch_t, wf1, bf1, g2, be2, wf2, bf2)
    return out


def kernel(x, edge_index, edge_attr, batch, W1l, W1r, b1, g1, be1,
           W2l, W2r, b2, Wf1, bf1, g2, be2, Wf2, bf2):
    edge4 = jnp.full((2, E_PAD), N_NODES, jnp.int32).at[:, :E_EDGES].set(
        edge_index.astype(jnp.int32)).reshape(
            2, N_TILES, CHUNKS, CHUNK).transpose(1, 0, 2, 3)
    batch_t = jnp.full((1, N_PAD), 64, jnp.int32).at[0, :N_NODES].set(
        batch.astype(jnp.int32))
    zero16 = jnp.zeros((N_PAD, 16), jnp.float32)

    table1, xr = _stage1(x, W1l, W1r)
    acc1, degp = _make_sc_agg_deg()(table1, edge4, zero16)
    table2, rest2, invdeg = _stage2(
        acc1[0], acc1[1], degp, xr, W2l, W2r,
        b1.reshape(1, 16), g1.reshape(1, 16), be1.reshape(1, 16),
        b2.reshape(1, 16))
    acc2 = _sc_agg(table2, edge4, zero16, 16)
    return _stage3(acc2[0], acc2[1], rest2, invdeg, batch_t,
                   Wf1, bf1.reshape(1, 32), g2.reshape(1, 32),
                   be2.reshape(1, 32), Wf2, bf2.reshape(1, 2))
